# baseline (speedup 1.0000x reference)
import math

import jax
import jax.numpy as jnp
from jax.experimental import pallas as pl


_N = 64
_D = 32
_H = 32
_K = 8
_TB = 8
_M = _TB * _N


def _gelu(x):
    return 0.5 * x * (1.0 + jax.lax.erf(x * (1.0 / math.sqrt(2.0))))


_C10 = (((1,), (0,)), ((), ()))
_C11 = (((1,), (1,)), ((), ()))
_BMM = (((2,), (2,)), ((0,), (0,)))


def _token_kernel(state_ref, tf_ref, sp_ref, tproj_ref, qw_ref, kw_ref,
                  qb_ref, kb_ref, wxi_ref, wxj_ref,
                  spA_ref, spC_ref, w2_ref, b2_ref, w3_ref,
                  wc0_ref, wc1_ref, wc2_ref, wc3_ref, wc4_ref,
                  mbias_ref, alpha_ref, r_ref,
                  lr_ref, attn_ref):
    f32 = jnp.float32
    N = _N
    K = _K
    M = _M
    qb = qb_ref[...]
    kb = kb_ref[...]
    wxi = wxi_ref[...]
    wxj = wxj_ref[...]
    w2 = w2_ref[...]
    b2 = b2_ref[...]
    w3 = w3_ref[...]
    alpha = alpha_ref[...]
    r_row = r_ref[...]

    jjf = jax.lax.broadcasted_iota(jnp.int32, (M, N), 1).astype(f32)

    def tile_tok(x):
        a, b = x.shape
        return jnp.broadcast_to(x.reshape(1, a, b), (_TB, a, b)).reshape(M, b)

    sp_t = tile_tok(sp_ref[...])
    spA_t = tile_tok(spA_ref[...])
    spC_t = tile_tok(spC_ref[...])
    wc0 = tile_tok(wc0_ref[...])
    wc1 = tile_tok(wc1_ref[...])
    wc2 = tile_tok(wc2_ref[...])
    wc3 = tile_tok(wc3_ref[...])
    wc4 = tile_tok(wc4_ref[...])
    mbias = tile_tok(mbias_ref[...])

    state_blk = state_ref[...]
    t0 = jax.lax.broadcasted_iota(jnp.int32, (_TB, _TB), 0)
    t1 = jax.lax.broadcasted_iota(jnp.int32, (_TB, _TB), 1)
    eyeT = jnp.where(t0 == t1, 1.0, 0.0).astype(f32)
    sT = jax.lax.dot_general(state_blk, eyeT, (((0,), (0,)), ((), ())),
                             preferred_element_type=f32)
    xi_b = jnp.concatenate(
        [jnp.broadcast_to(sT[:, t:t + 1], (N, N)) for t in range(_TB)],
        axis=0)
    s_col = xi_b[:, 0:1]
    xj_b = jnp.broadcast_to(state_blk.reshape(_TB, 1, N),
                            (_TB, N, N)).reshape(M, N)

    tf = tf_ref[...].reshape(M, _D)
    proj = jax.lax.dot_general(tf + sp_t, tproj_ref[...], _C10,
                               preferred_element_type=f32)
    feats = jnp.concatenate([s_col, proj], axis=1)
    q = jax.lax.dot_general(feats, qw_ref[...], _C10,
                            preferred_element_type=f32) + qb
    k = jax.lax.dot_general(feats, kw_ref[...], _C10,
                            preferred_element_type=f32) + kb
    scores = jax.lax.dot_general(q.reshape(_TB, N, _D), k.reshape(_TB, N, _D),
                                 _BMM, preferred_element_type=f32)
    scores = (scores / (_D ** 0.5)).reshape(M, N)

    removed = jnp.zeros((M, N), jnp.bool_)
    firsts = []
    for _ in range(K):
        masked = jnp.where(removed, -jnp.inf, scores)
        m = jnp.max(masked, axis=1, keepdims=True)
        sel = masked == m
        minidx = jnp.min(jnp.where(sel, jjf, float(N)), axis=1, keepdims=True)
        first = sel & (jjf == minidx)
        removed = removed | first
        firsts.append(jnp.where(first, 1.0, 0.0).astype(f32))
    keep = removed

    S3 = jnp.concatenate([f.reshape(M, 1, N) for f in firsts],
                         axis=1)

    A = s_col * wxi + spA_t
    C = s_col * wxj + spC_t
    C_sel = jax.lax.dot_general(S3.reshape(_TB, N * K, N),
                                C.reshape(_TB, N, _H),
                                (((2,), (1,)), ((0,), (0,))),
                                preferred_element_type=f32)
    A_sel = jnp.broadcast_to(A.reshape(M, 1, _H), (M, K, _H))
    h1 = _gelu(A_sel.reshape(M * K, _H) + C_sel.reshape(M * K, _H))
    h2 = _gelu(jax.lax.dot_general(h1, w2, _C10,
                                   preferred_element_type=f32) + b2)
    f4s = jnp.sum(h2.reshape(M, K, _H) * w3.reshape(1, 1, _H),
                  axis=2)

    f4d = firsts[0] * f4s[:, 0:1]
    for rr in range(1, K):
        f4d = f4d + firsts[rr] * f4s[:, rr:rr + 1]

    holl = xj_b / (1.0 + alpha * xj_b)
    msgs = (wc0 * xj_b + wc1 * xi_b * xj_b + wc2 * holl +
            wc3 * xi_b * holl + wc4 * f4d + mbias)

    rowmax = jnp.max(scores, axis=1, keepdims=True)
    e = jnp.where(keep, jnp.exp(scores - rowmax), 0.0)
    z = jnp.sum(e, axis=1, keepdims=True)
    attn = e / z

    agg = jnp.sum((attn * msgs).reshape(_TB, N, N), axis=2)
    lr_ref[...] = r_row + agg
    attn_ref[...] = attn.reshape(_TB, N, N)


def _rep(shape):
    nd = len(shape)
    return pl.BlockSpec(shape, lambda i, _nd=nd: (0,) * _nd)


def kernel(state, temporal_feat, species_emb, q_W, q_b, k_W, k_b, tproj_W,
           form_coefs, form_gates_raw, holling_alpha_raw,
           mlp_W1, mlp_b1, mlp_W2, mlp_b2, mlp_W3, mlp_b3, r):
    B, T, N = state.shape
    D = species_emb.shape[1]
    H = mlp_W2.shape[0]
    BT = B * T

    gates = jax.nn.sigmoid(form_gates_raw)
    wc = form_coefs * gates
    alpha = (jax.nn.softplus(holling_alpha_raw) + 0.01).reshape(1, N)
    spA = species_emb @ mlp_W1[2:2 + D] + mlp_b1
    spC = species_emb @ mlp_W1[2 + D:2 + 2 * D]
    wxi = mlp_W1[0].reshape(1, H)
    wxj = mlp_W1[1].reshape(1, H)
    mbias = wc[4] * mlp_b3[0]
    w3 = mlp_W3.reshape(1, H)

    state2 = state.reshape(BT, N)
    tf2 = temporal_feat.reshape(BT, N, D)

    grid = (BT // _TB,)
    out_shape = (
        jax.ShapeDtypeStruct((BT, N), jnp.float32),
        jax.ShapeDtypeStruct((BT, N, N), jnp.float32),
    )
    in_specs = [
        pl.BlockSpec((_TB, N), lambda i: (i, 0)),
        pl.BlockSpec((_TB, N, D), lambda i: (i, 0, 0)),
        _rep((N, D)),
        _rep((D, D)),
        _rep((1 + D, D)),
        _rep((1 + D, D)),
        _rep((1, D)),
        _rep((1, D)),
        _rep((1, H)),
        _rep((1, H)),
        _rep((N, H)),
        _rep((N, H)),
        _rep((H, H)),
        _rep((1, H)),
        _rep((1, H)),
        _rep((N, N)),
        _rep((N, N)),
        _rep((N, N)),
        _rep((N, N)),
        _rep((N, N)),
        _rep((N, N)),
        _rep((1, N)),
        _rep((1, N)),
    ]
    out_specs = (
        pl.BlockSpec((_TB, N), lambda i: (i, 0)),
        pl.BlockSpec((_TB, N, N), lambda i: (i, 0, 0)),
    )

    lr2, attn2 = pl.pallas_call(
        _token_kernel,
        grid=grid,
        in_specs=in_specs,
        out_specs=out_specs,
        out_shape=out_shape,
    )(state2, tf2, species_emb, tproj_W, q_W, k_W,
      q_b.reshape(1, D), k_b.reshape(1, D), wxi, wxj, spA, spC,
      mlp_W2, mlp_b2.reshape(1, H), w3,
      wc[0], wc[1], wc[2], wc[3], wc[4], mbias, alpha, r.reshape(1, N))

    return lr2.reshape(B, T, N), attn2.reshape(B, T, N, N)


# speedup vs baseline: 1.4027x; 1.4027x over previous
import math

import jax
import jax.numpy as jnp
from jax.experimental import pallas as pl


_N = 64
_D = 32
_H = 32
_K = 8
_TB = 8
_M = _TB * _N


def _gelu(x):
    return 0.5 * x * (1.0 + jax.lax.erf(x * (1.0 / math.sqrt(2.0))))


_C10 = (((1,), (0,)), ((), ()))
_C11 = (((1,), (1,)), ((), ()))
_BMM = (((2,), (2,)), ((0,), (0,)))


def _token_kernel(state_ref, tf_ref, sp_ref, tproj_ref, qw_ref, kw_ref,
                  qb_ref, kb_ref, wxi_ref, wxj_ref,
                  spA_ref, spC_ref, w2_ref, b2_ref, w3_ref,
                  wc0_ref, wc1_ref, wc2_ref, wc3_ref, wc4_ref,
                  mbias_ref, alpha_ref, r_ref,
                  lr_ref, attn_ref):
    f32 = jnp.float32
    N = _N
    K = _K
    M = _M
    qb = qb_ref[...]
    kb = kb_ref[...]
    wxi = wxi_ref[...]
    wxj = wxj_ref[...]
    w2 = w2_ref[...]
    b2 = b2_ref[...]
    w3 = w3_ref[...]
    alpha = alpha_ref[...]
    r_row = r_ref[...]

    i0 = jax.lax.broadcasted_iota(jnp.int32, (N, N), 0)
    i1 = jax.lax.broadcasted_iota(jnp.int32, (N, N), 1)
    cumU = jnp.where(i1 <= i0, 1.0, 0.0).astype(f32)
    jjf = jax.lax.broadcasted_iota(jnp.int32, (M, N), 1).astype(f32)
    slot_i = jax.lax.broadcasted_iota(jnp.int32, (M, K, N), 1)

    def tile_tok(x):
        a, b = x.shape
        return jnp.broadcast_to(x.reshape(1, a, b), (_TB, a, b)).reshape(M, b)

    sp_t = tile_tok(sp_ref[...])
    spA_t = tile_tok(spA_ref[...])
    spC_t = tile_tok(spC_ref[...])
    wc0 = tile_tok(wc0_ref[...])
    wc1 = tile_tok(wc1_ref[...])
    wc2 = tile_tok(wc2_ref[...])
    wc3 = tile_tok(wc3_ref[...])
    wc4 = tile_tok(wc4_ref[...])
    mbias = tile_tok(mbias_ref[...])

    state_blk = state_ref[...]
    t0 = jax.lax.broadcasted_iota(jnp.int32, (_TB, _TB), 0)
    t1 = jax.lax.broadcasted_iota(jnp.int32, (_TB, _TB), 1)
    eyeT = jnp.where(t0 == t1, 1.0, 0.0).astype(f32)
    sT = jax.lax.dot_general(state_blk, eyeT, (((0,), (0,)), ((), ())),
                             preferred_element_type=f32)
    xi_b = jnp.concatenate(
        [jnp.broadcast_to(sT[:, t:t + 1], (N, N)) for t in range(_TB)],
        axis=0)
    s_col = xi_b[:, 0:1]
    xj_b = jnp.broadcast_to(state_blk.reshape(_TB, 1, N),
                            (_TB, N, N)).reshape(M, N)

    tf = tf_ref[...].reshape(M, _D)
    proj = jax.lax.dot_general(tf + sp_t, tproj_ref[...], _C10,
                               preferred_element_type=f32)
    feats = jnp.concatenate([s_col, proj], axis=1)
    q = jax.lax.dot_general(feats, qw_ref[...], _C10,
                            preferred_element_type=f32) + qb
    k = jax.lax.dot_general(feats, kw_ref[...], _C10,
                            preferred_element_type=f32) + kb
    scores = jax.lax.dot_general(q.reshape(_TB, N, _D), k.reshape(_TB, N, _D),
                                 _BMM, preferred_element_type=f32)
    scores = (scores / (_D ** 0.5)).reshape(M, N)

    cur = scores
    for _ in range(K):
        m = jnp.max(cur, axis=1, keepdims=True)
        idxm = jnp.where(cur == m, jjf, float(N))
        minidx = jnp.min(idxm, axis=1, keepdims=True)
        cur = jnp.where(idxm == minidx, -jnp.inf, cur)
    keep = cur == -jnp.inf
    keepf = jnp.where(keep, 1.0, 0.0).astype(f32)

    kcum = jax.lax.dot_general(keepf, cumU, _C11,
                               preferred_element_type=f32)
    slotv = (kcum - 0.5).astype(jnp.int32).reshape(M, 1, N)
    keep3 = keepf.reshape(M, 1, N)
    S3 = jnp.where((jnp.broadcast_to(slotv, (M, K, N)) == slot_i) &
                   (jnp.broadcast_to(keep3, (M, K, N)) > 0.0),
                   1.0, 0.0).astype(f32)

    A = s_col * wxi + spA_t
    C = s_col * wxj + spC_t
    C_sel = jax.lax.dot_general(S3.reshape(_TB, N * K, N),
                                C.reshape(_TB, N, _H),
                                (((2,), (1,)), ((0,), (0,))),
                                preferred_element_type=f32)
    A_sel = jnp.broadcast_to(A.reshape(M, 1, _H), (M, K, _H))
    h1 = _gelu(A_sel.reshape(M * K, _H) + C_sel.reshape(M * K, _H))
    h2 = _gelu(jax.lax.dot_general(h1, w2, _C10,
                                   preferred_element_type=f32) + b2)
    f4s = jnp.sum(h2.reshape(M, K, _H) * w3.reshape(1, 1, _H),
                  axis=2)

    f4d = jnp.sum(S3 * f4s.reshape(M, K, 1), axis=1)

    holl = xj_b / (1.0 + alpha * xj_b)
    msgs = (wc0 * xj_b + wc1 * xi_b * xj_b + wc2 * holl +
            wc3 * xi_b * holl + wc4 * f4d + mbias)

    rowmax = jnp.max(scores, axis=1, keepdims=True)
    e = jnp.where(keep, jnp.exp(scores - rowmax), 0.0)
    z = jnp.sum(e, axis=1, keepdims=True)
    attn = e / z

    agg = jnp.sum((attn * msgs).reshape(_TB, N, N), axis=2)
    lr_ref[...] = r_row + agg
    attn_ref[...] = attn.reshape(_TB, N, N)


def _rep(shape):
    nd = len(shape)
    return pl.BlockSpec(shape, lambda i, _nd=nd: (0,) * _nd)


def kernel(state, temporal_feat, species_emb, q_W, q_b, k_W, k_b, tproj_W,
           form_coefs, form_gates_raw, holling_alpha_raw,
           mlp_W1, mlp_b1, mlp_W2, mlp_b2, mlp_W3, mlp_b3, r):
    B, T, N = state.shape
    D = species_emb.shape[1]
    H = mlp_W2.shape[0]
    BT = B * T

    gates = jax.nn.sigmoid(form_gates_raw)
    wc = form_coefs * gates
    alpha = (jax.nn.softplus(holling_alpha_raw) + 0.01).reshape(1, N)
    spA = species_emb @ mlp_W1[2:2 + D] + mlp_b1
    spC = species_emb @ mlp_W1[2 + D:2 + 2 * D]
    wxi = mlp_W1[0].reshape(1, H)
    wxj = mlp_W1[1].reshape(1, H)
    mbias = wc[4] * mlp_b3[0]
    w3 = mlp_W3.reshape(1, H)

    state2 = state.reshape(BT, N)
    tf2 = temporal_feat.reshape(BT, N, D)

    grid = (BT // _TB,)
    out_shape = (
        jax.ShapeDtypeStruct((BT, N), jnp.float32),
        jax.ShapeDtypeStruct((BT, N, N), jnp.float32),
    )
    in_specs = [
        pl.BlockSpec((_TB, N), lambda i: (i, 0)),
        pl.BlockSpec((_TB, N, D), lambda i: (i, 0, 0)),
        _rep((N, D)),
        _rep((D, D)),
        _rep((1 + D, D)),
        _rep((1 + D, D)),
        _rep((1, D)),
        _rep((1, D)),
        _rep((1, H)),
        _rep((1, H)),
        _rep((N, H)),
        _rep((N, H)),
        _rep((H, H)),
        _rep((1, H)),
        _rep((1, H)),
        _rep((N, N)),
        _rep((N, N)),
        _rep((N, N)),
        _rep((N, N)),
        _rep((N, N)),
        _rep((N, N)),
        _rep((1, N)),
        _rep((1, N)),
    ]
    out_specs = (
        pl.BlockSpec((_TB, N), lambda i: (i, 0)),
        pl.BlockSpec((_TB, N, N), lambda i: (i, 0, 0)),
    )

    lr2, attn2 = pl.pallas_call(
        _token_kernel,
        grid=grid,
        in_specs=in_specs,
        out_specs=out_specs,
        out_shape=out_shape,
    )(state2, tf2, species_emb, tproj_W, q_W, k_W,
      q_b.reshape(1, D), k_b.reshape(1, D), wxi, wxj, spA, spC,
      mlp_W2, mlp_b2.reshape(1, H), w3,
      wc[0], wc[1], wc[2], wc[3], wc[4], mbias, alpha, r.reshape(1, N))

    return lr2.reshape(B, T, N), attn2.reshape(B, T, N, N)


# speedup vs baseline: 1.5241x; 1.0865x over previous
import math

import jax
import jax.numpy as jnp
from jax.experimental import pallas as pl


_N = 64
_D = 32
_H = 32
_K = 8
_TB = 16
_M = _TB * _N


def _gelu(x):
    return 0.5 * x * (1.0 + jax.lax.erf(x * (1.0 / math.sqrt(2.0))))


_C10 = (((1,), (0,)), ((), ()))
_C11 = (((1,), (1,)), ((), ()))
_BMM = (((2,), (2,)), ((0,), (0,)))


def _token_kernel(state_ref, tf_ref, sp_ref, tproj_ref, qw_ref, kw_ref,
                  qb_ref, kb_ref, wxi_ref, wxj_ref,
                  spA_ref, spC_ref, w2_ref, b2_ref, w3_ref,
                  wc0_ref, wc1_ref, wc2_ref, wc3_ref, wc4_ref,
                  mbias_ref, alpha_ref, r_ref,
                  lr_ref, attn_ref):
    f32 = jnp.float32
    N = _N
    K = _K
    M = _M
    qb = qb_ref[...]
    kb = kb_ref[...]
    wxi = wxi_ref[...]
    wxj = wxj_ref[...]
    w2 = w2_ref[...]
    b2 = b2_ref[...]
    w3 = w3_ref[...]
    alpha = alpha_ref[...]
    r_row = r_ref[...]

    i0 = jax.lax.broadcasted_iota(jnp.int32, (N, N), 0)
    i1 = jax.lax.broadcasted_iota(jnp.int32, (N, N), 1)
    cumU = jnp.where(i1 <= i0, 1.0, 0.0).astype(f32)
    jjf = jax.lax.broadcasted_iota(jnp.int32, (M, N), 1).astype(f32)
    slot_i = jax.lax.broadcasted_iota(jnp.int32, (M, K, N), 1)

    def tile_tok(x):
        a, b = x.shape
        return jnp.broadcast_to(x.reshape(1, a, b), (_TB, a, b)).reshape(M, b)

    sp_t = tile_tok(sp_ref[...])
    spA_t = tile_tok(spA_ref[...])
    spC_t = tile_tok(spC_ref[...])
    wc0 = tile_tok(wc0_ref[...])
    wc1 = tile_tok(wc1_ref[...])
    wc2 = tile_tok(wc2_ref[...])
    wc3 = tile_tok(wc3_ref[...])
    wc4 = tile_tok(wc4_ref[...])
    mbias = tile_tok(mbias_ref[...])

    state_blk = state_ref[...]
    t0 = jax.lax.broadcasted_iota(jnp.int32, (_TB, _TB), 0)
    t1 = jax.lax.broadcasted_iota(jnp.int32, (_TB, _TB), 1)
    eyeT = jnp.where(t0 == t1, 1.0, 0.0).astype(f32)
    sT = jax.lax.dot_general(state_blk, eyeT, (((0,), (0,)), ((), ())),
                             preferred_element_type=f32)
    xi_b = jnp.concatenate(
        [jnp.broadcast_to(sT[:, t:t + 1], (N, N)) for t in range(_TB)],
        axis=0)
    s_col = xi_b[:, 0:1]
    xj_b = jnp.broadcast_to(state_blk.reshape(_TB, 1, N),
                            (_TB, N, N)).reshape(M, N)

    tf = tf_ref[...].reshape(M, _D)
    proj = jax.lax.dot_general(tf + sp_t, tproj_ref[...], _C10,
                               preferred_element_type=f32)
    feats = jnp.concatenate([s_col, proj], axis=1)
    q = jax.lax.dot_general(feats, qw_ref[...], _C10,
                            preferred_element_type=f32) + qb
    k = jax.lax.dot_general(feats, kw_ref[...], _C10,
                            preferred_element_type=f32) + kb
    scores = jax.lax.dot_general(q.reshape(_TB, N, _D), k.reshape(_TB, N, _D),
                                 _BMM, preferred_element_type=f32)
    scores = (scores / (_D ** 0.5)).reshape(M, N)

    cur = scores
    for _ in range(K):
        m = jnp.max(cur, axis=1, keepdims=True)
        idxm = jnp.where(cur == m, jjf, float(N))
        minidx = jnp.min(idxm, axis=1, keepdims=True)
        cur = jnp.where(idxm == minidx, -jnp.inf, cur)
    keep = cur == -jnp.inf
    keepf = jnp.where(keep, 1.0, 0.0).astype(f32)

    kcum = jax.lax.dot_general(keepf, cumU, _C11,
                               preferred_element_type=f32)
    slotv = (kcum - 0.5).astype(jnp.int32).reshape(M, 1, N)
    keep3 = keepf.reshape(M, 1, N)
    S3 = jnp.where((jnp.broadcast_to(slotv, (M, K, N)) == slot_i) &
                   (jnp.broadcast_to(keep3, (M, K, N)) > 0.0),
                   1.0, 0.0).astype(f32)

    A = s_col * wxi + spA_t
    C = s_col * wxj + spC_t
    C_sel = jax.lax.dot_general(S3.reshape(_TB, N * K, N),
                                C.reshape(_TB, N, _H),
                                (((2,), (1,)), ((0,), (0,))),
                                preferred_element_type=f32)
    A_sel = jnp.broadcast_to(A.reshape(M, 1, _H), (M, K, _H))
    h1 = _gelu(A_sel.reshape(M * K, _H) + C_sel.reshape(M * K, _H))
    h2 = _gelu(jax.lax.dot_general(h1, w2, _C10,
                                   preferred_element_type=f32) + b2)
    f4s = jnp.sum(h2.reshape(M, K, _H) * w3.reshape(1, 1, _H),
                  axis=2)

    f4d = jnp.sum(S3 * f4s.reshape(M, K, 1), axis=1)

    holl = xj_b / (1.0 + alpha * xj_b)
    msgs = (wc0 * xj_b + wc1 * xi_b * xj_b + wc2 * holl +
            wc3 * xi_b * holl + wc4 * f4d + mbias)

    rowmax = jnp.max(scores, axis=1, keepdims=True)
    e = jnp.where(keep, jnp.exp(scores - rowmax), 0.0)
    z = jnp.sum(e, axis=1, keepdims=True)
    attn = e / z

    agg = jnp.sum((attn * msgs).reshape(_TB, N, N), axis=2)
    lr_ref[...] = r_row + agg
    attn_ref[...] = attn.reshape(_TB, N, N)


def _rep(shape):
    nd = len(shape)
    return pl.BlockSpec(shape, lambda i, _nd=nd: (0,) * _nd)


def kernel(state, temporal_feat, species_emb, q_W, q_b, k_W, k_b, tproj_W,
           form_coefs, form_gates_raw, holling_alpha_raw,
           mlp_W1, mlp_b1, mlp_W2, mlp_b2, mlp_W3, mlp_b3, r):
    B, T, N = state.shape
    D = species_emb.shape[1]
    H = mlp_W2.shape[0]
    BT = B * T

    gates = jax.nn.sigmoid(form_gates_raw)
    wc = form_coefs * gates
    alpha = (jax.nn.softplus(holling_alpha_raw) + 0.01).reshape(1, N)
    spA = species_emb @ mlp_W1[2:2 + D] + mlp_b1
    spC = species_emb @ mlp_W1[2 + D:2 + 2 * D]
    wxi = mlp_W1[0].reshape(1, H)
    wxj = mlp_W1[1].reshape(1, H)
    mbias = wc[4] * mlp_b3[0]
    w3 = mlp_W3.reshape(1, H)

    state2 = state.reshape(BT, N)
    tf2 = temporal_feat.reshape(BT, N, D)

    grid = (BT // _TB,)
    out_shape = (
        jax.ShapeDtypeStruct((BT, N), jnp.float32),
        jax.ShapeDtypeStruct((BT, N, N), jnp.float32),
    )
    in_specs = [
        pl.BlockSpec((_TB, N), lambda i: (i, 0)),
        pl.BlockSpec((_TB, N, D), lambda i: (i, 0, 0)),
        _rep((N, D)),
        _rep((D, D)),
        _rep((1 + D, D)),
        _rep((1 + D, D)),
        _rep((1, D)),
        _rep((1, D)),
        _rep((1, H)),
        _rep((1, H)),
        _rep((N, H)),
        _rep((N, H)),
        _rep((H, H)),
        _rep((1, H)),
        _rep((1, H)),
        _rep((N, N)),
        _rep((N, N)),
        _rep((N, N)),
        _rep((N, N)),
        _rep((N, N)),
        _rep((N, N)),
        _rep((1, N)),
        _rep((1, N)),
    ]
    out_specs = (
        pl.BlockSpec((_TB, N), lambda i: (i, 0)),
        pl.BlockSpec((_TB, N, N), lambda i: (i, 0, 0)),
    )

    lr2, attn2 = pl.pallas_call(
        _token_kernel,
        grid=grid,
        in_specs=in_specs,
        out_specs=out_specs,
        out_shape=out_shape,
    )(state2, tf2, species_emb, tproj_W, q_W, k_W,
      q_b.reshape(1, D), k_b.reshape(1, D), wxi, wxj, spA, spC,
      mlp_W2, mlp_b2.reshape(1, H), w3,
      wc[0], wc[1], wc[2], wc[3], wc[4], mbias, alpha, r.reshape(1, N))

    return lr2.reshape(B, T, N), attn2.reshape(B, T, N, N)


# speedup vs baseline: 1.5277x; 1.0023x over previous
import math

import jax
import jax.numpy as jnp
from jax.experimental import pallas as pl


_N = 64
_D = 32
_H = 32
_K = 8
_TB = 16
_M = _TB * _N


def _gelu(x):
    return 0.5 * x * (1.0 + jax.lax.erf(x * (1.0 / math.sqrt(2.0))))


_C10 = (((1,), (0,)), ((), ()))
_C11 = (((1,), (1,)), ((), ()))
_BMM = (((2,), (2,)), ((0,), (0,)))


def _token_kernel(state_ref, tf_ref, sp_ref, tproj_ref, qw_ref, kw_ref,
                  qb_ref, kb_ref, wxi_ref, wxj_ref,
                  spA_ref, spC_ref, w2_ref, b2_ref, w3_ref,
                  wc0_ref, wc1_ref, wc2_ref, wc3_ref, wc4_ref,
                  mbias_ref, alpha_ref, r_ref,
                  lr_ref, attn_ref):
    f32 = jnp.float32
    N = _N
    K = _K
    M = _M
    qb = qb_ref[...]
    kb = kb_ref[...]
    wxi = wxi_ref[...]
    wxj = wxj_ref[...]
    w2 = w2_ref[...]
    b2 = b2_ref[...]
    w3 = w3_ref[...]
    alpha = alpha_ref[...]
    r_row = r_ref[...]

    i0 = jax.lax.broadcasted_iota(jnp.int32, (N, N), 0)
    i1 = jax.lax.broadcasted_iota(jnp.int32, (N, N), 1)
    cumU = jnp.where(i1 <= i0, 1.0, 0.0).astype(f32)
    jjf = jax.lax.broadcasted_iota(jnp.int32, (M, N), 1).astype(f32)
    slot_i = jax.lax.broadcasted_iota(jnp.int32, (M, K, N), 1)

    def tile_tok(x):
        a, b = x.shape
        return jnp.broadcast_to(x.reshape(1, a, b), (_TB, a, b)).reshape(M, b)

    spA_t = tile_tok(spA_ref[...])
    spC_t = tile_tok(spC_ref[...])
    sp3 = sp_ref[...].reshape(1, N, _D)
    wc0 = wc0_ref[...].reshape(1, N, N)
    wc1 = wc1_ref[...].reshape(1, N, N)
    wc2 = wc2_ref[...].reshape(1, N, N)
    wc3 = wc3_ref[...].reshape(1, N, N)
    wc4 = wc4_ref[...].reshape(1, N, N)
    mbias = mbias_ref[...].reshape(1, N, N)

    state_blk = state_ref[...]
    t0 = jax.lax.broadcasted_iota(jnp.int32, (_TB, _TB), 0)
    t1 = jax.lax.broadcasted_iota(jnp.int32, (_TB, _TB), 1)
    eyeT = jnp.where(t0 == t1, 1.0, 0.0).astype(f32)
    sT = jax.lax.dot_general(state_blk, eyeT, (((0,), (0,)), ((), ())),
                             preferred_element_type=f32)
    xi_b = jnp.concatenate(
        [jnp.broadcast_to(sT[:, t:t + 1], (N, N)) for t in range(_TB)],
        axis=0)
    s_col = xi_b[:, 0:1]
    xj3 = jnp.broadcast_to(state_blk.reshape(_TB, 1, N),
                           (_TB, N, N))
    xi3 = xi_b.reshape(_TB, N, N)

    tf = (tf_ref[...] + sp3).reshape(M, _D)
    proj = jax.lax.dot_general(tf, tproj_ref[...], _C10,
                               preferred_element_type=f32)
    feats = jnp.concatenate([s_col, proj], axis=1)
    q = jax.lax.dot_general(feats, qw_ref[...], _C10,
                            preferred_element_type=f32) + qb
    k = jax.lax.dot_general(feats, kw_ref[...], _C10,
                            preferred_element_type=f32) + kb
    scores = jax.lax.dot_general(q.reshape(_TB, N, _D), k.reshape(_TB, N, _D),
                                 _BMM, preferred_element_type=f32)
    scores = (scores / (_D ** 0.5)).reshape(M, N)

    cur = scores
    for _ in range(K):
        m = jnp.max(cur, axis=1, keepdims=True)
        idxm = jnp.where(cur == m, jjf, float(N))
        minidx = jnp.min(idxm, axis=1, keepdims=True)
        cur = jnp.where(idxm == minidx, -jnp.inf, cur)
    keep = cur == -jnp.inf
    keepf = jnp.where(keep, 1.0, 0.0).astype(f32)

    kcum = jax.lax.dot_general(keepf, cumU, _C11,
                               preferred_element_type=f32)
    slotv = (kcum - 0.5).astype(jnp.int32).reshape(M, 1, N)
    keep3 = keepf.reshape(M, 1, N)
    S3 = jnp.where((jnp.broadcast_to(slotv, (M, K, N)) == slot_i) &
                   (jnp.broadcast_to(keep3, (M, K, N)) > 0.0),
                   1.0, 0.0).astype(f32)

    A = s_col * wxi + spA_t
    C = s_col * wxj + spC_t
    C_sel = jax.lax.dot_general(S3.reshape(_TB, N * K, N),
                                C.reshape(_TB, N, _H),
                                (((2,), (1,)), ((0,), (0,))),
                                preferred_element_type=f32)
    A_sel = jnp.broadcast_to(A.reshape(M, 1, _H), (M, K, _H))
    h1 = _gelu(A_sel.reshape(M * K, _H) + C_sel.reshape(M * K, _H))
    h2 = _gelu(jax.lax.dot_general(h1, w2, _C10,
                                   preferred_element_type=f32) + b2)
    f4s = jnp.sum(h2.reshape(M, K, _H) * w3.reshape(1, 1, _H),
                  axis=2)

    f4d = jnp.sum(S3 * f4s.reshape(M, K, 1), axis=1)

    alpha3 = alpha.reshape(1, 1, N)
    holl = xj3 / (1.0 + alpha3 * xj3)
    msgs = (wc0 * xj3 + wc1 * xi3 * xj3 + wc2 * holl +
            wc3 * xi3 * holl + wc4 * f4d.reshape(_TB, N, N) +
            mbias)

    rowmax = jnp.max(scores, axis=1, keepdims=True)
    e = jnp.where(keep, jnp.exp(scores - rowmax), 0.0)
    z = jnp.sum(e, axis=1, keepdims=True)
    attn = (e / z).reshape(_TB, N, N)

    agg = jnp.sum(attn * msgs, axis=2)
    lr_ref[...] = r_row + agg
    attn_ref[...] = attn


def _rep(shape):
    nd = len(shape)
    return pl.BlockSpec(shape, lambda i, _nd=nd: (0,) * _nd)


def kernel(state, temporal_feat, species_emb, q_W, q_b, k_W, k_b, tproj_W,
           form_coefs, form_gates_raw, holling_alpha_raw,
           mlp_W1, mlp_b1, mlp_W2, mlp_b2, mlp_W3, mlp_b3, r):
    B, T, N = state.shape
    D = species_emb.shape[1]
    H = mlp_W2.shape[0]
    BT = B * T

    gates = jax.nn.sigmoid(form_gates_raw)
    wc = form_coefs * gates
    alpha = (jax.nn.softplus(holling_alpha_raw) + 0.01).reshape(1, N)
    spA = species_emb @ mlp_W1[2:2 + D] + mlp_b1
    spC = species_emb @ mlp_W1[2 + D:2 + 2 * D]
    wxi = mlp_W1[0].reshape(1, H)
    wxj = mlp_W1[1].reshape(1, H)
    mbias = wc[4] * mlp_b3[0]
    w3 = mlp_W3.reshape(1, H)

    state2 = state.reshape(BT, N)
    tf2 = temporal_feat.reshape(BT, N, D)

    grid = (BT // _TB,)
    out_shape = (
        jax.ShapeDtypeStruct((BT, N), jnp.float32),
        jax.ShapeDtypeStruct((BT, N, N), jnp.float32),
    )
    in_specs = [
        pl.BlockSpec((_TB, N), lambda i: (i, 0)),
        pl.BlockSpec((_TB, N, D), lambda i: (i, 0, 0)),
        _rep((N, D)),
        _rep((D, D)),
        _rep((1 + D, D)),
        _rep((1 + D, D)),
        _rep((1, D)),
        _rep((1, D)),
        _rep((1, H)),
        _rep((1, H)),
        _rep((N, H)),
        _rep((N, H)),
        _rep((H, H)),
        _rep((1, H)),
        _rep((1, H)),
        _rep((N, N)),
        _rep((N, N)),
        _rep((N, N)),
        _rep((N, N)),
        _rep((N, N)),
        _rep((N, N)),
        _rep((1, N)),
        _rep((1, N)),
    ]
    out_specs = (
        pl.BlockSpec((_TB, N), lambda i: (i, 0)),
        pl.BlockSpec((_TB, N, N), lambda i: (i, 0, 0)),
    )

    lr2, attn2 = pl.pallas_call(
        _token_kernel,
        grid=grid,
        in_specs=in_specs,
        out_specs=out_specs,
        out_shape=out_shape,
    )(state2, tf2, species_emb, tproj_W, q_W, k_W,
      q_b.reshape(1, D), k_b.reshape(1, D), wxi, wxj, spA, spC,
      mlp_W2, mlp_b2.reshape(1, H), w3,
      wc[0], wc[1], wc[2], wc[3], wc[4], mbias, alpha, r.reshape(1, N))

    return lr2.reshape(B, T, N), attn2.reshape(B, T, N, N)
